# Initial kernel scaffold; baseline (speedup 1.0000x reference)
import jax
import jax.numpy as jnp
from jax.experimental import pallas as pl

_TIME_STEPS = 32
_TAU = 10.0
_B_BLK = 32


def _body(x_ref, out_ref):
    x = x_ref[...]
    s = jax.nn.sigmoid(x) + 1e-07
    latency = -_TAU * jnp.log(s)
    t = jnp.floor(jnp.clip(latency, 0.0, float(_TIME_STEPS - 1)))
    tt = t[:, None, :]
    time_iota = jax.lax.broadcasted_iota(
        jnp.float32, (x.shape[0], _TIME_STEPS, x.shape[1]), 1
    )
    out_ref[...] = jnp.where(tt == time_iota, 1.0, 0.0).astype(jnp.float32)


def kernel(x):
    B, I = x.shape
    grid = (B // _B_BLK,)
    return pl.pallas_call(
        _body,
        grid=grid,
        in_specs=[pl.BlockSpec((_B_BLK, I), lambda i: (i, 0))],
        out_specs=pl.BlockSpec((_B_BLK, _TIME_STEPS, I), lambda i: (i, 0, 0)),
        out_shape=jax.ShapeDtypeStruct((B, _TIME_STEPS, I), jnp.float32),
    )(x)


# speedup vs baseline: 101.9664x; 101.9664x over previous
import jax
import jax.numpy as jnp
from jax.experimental import pallas as pl

_TIME_STEPS = 32
_TAU = 10.0
_B_BLK = 32


def _body(x_ref, out_ref):
    x = x_ref[...]
    s = jax.nn.sigmoid(x) + 1e-07
    latency = -_TAU * jnp.log(s)
    t = jnp.clip(latency, 0.0, float(_TIME_STEPS - 1)).astype(jnp.int32)
    tt = t[:, None, :]
    time_iota = jax.lax.broadcasted_iota(
        jnp.int32, (x.shape[0], _TIME_STEPS, x.shape[1]), 1
    )
    out_ref[...] = jnp.where(tt == time_iota, 1.0, 0.0).astype(jnp.float32)


def kernel(x):
    B, I = x.shape
    grid = (B // _B_BLK,)
    return pl.pallas_call(
        _body,
        grid=grid,
        in_specs=[pl.BlockSpec((_B_BLK, I), lambda i: (i, 0))],
        out_specs=pl.BlockSpec((_B_BLK, _TIME_STEPS, I), lambda i: (i, 0, 0)),
        out_shape=jax.ShapeDtypeStruct((B, _TIME_STEPS, I), jnp.float32),
    )(x)
